# baseline (speedup 1.0000x reference)
import functools

import jax
import jax.numpy as jnp
from jax import lax
from jax.experimental import pallas as pl
from jax.experimental.pallas import tpu as pltpu
from jax.experimental.pallas import tpu_sc as plsc

VOCAB = 1000000
EMBED = 32
B_TOTAL = 16384 * 50

_info = plsc.get_sparse_core_info()
_NC, _NS = _info.num_cores, _info.num_subcores
_NW = _NC * _NS
_B_PER_W = B_TOTAL // _NW
_CHUNK = 800
_NCHUNK = _B_PER_W // _CHUNK
_NBUF = 4
_GDEPTH = 2
_NIBUF = 4


def _make_gather():
  mesh = plsc.VectorSubcoreMesh(core_axis_name="c", subcore_axis_name="s")

  @functools.partial(
      pl.kernel,
      mesh=mesh,
      out_type=jax.ShapeDtypeStruct((B_TOTAL, EMBED), jnp.float32),
      scratch_types=[
          pltpu.VMEM((_NIBUF, _CHUNK), jnp.int32),
          pltpu.VMEM((_NBUF, _CHUNK, EMBED), jnp.float32),
      ] + [pltpu.SemaphoreType.DMA] * (_NIBUF + 2 * _NBUF),
      compiler_params=pltpu.CompilerParams(use_tc_tiling_on_sc=False),
  )
  def gather_k(table_hbm, idx_hbm, out_hbm, idx_v, rows_v, *sems):
    isem = list(sems[:_NIBUF])
    gsem = list(sems[_NIBUF:_NIBUF + _NBUF])
    ssem = list(sems[_NIBUF + _NBUF:])
    wid = lax.axis_index("s") * _NC + lax.axis_index("c")
    base = wid * _B_PER_W

    idxd = [None] * _NCHUNK
    gd = [None] * _NCHUNK
    sd = [None] * _NCHUNK

    def start_idx(c):
      i = c % _NIBUF
      idxd[c] = pltpu.async_copy(
          idx_hbm.at[pl.ds(base + c * _CHUNK, _CHUNK)], idx_v.at[i],
          isem[i])

    def start_store(c):
      b = c % _NBUF
      sd[c] = pltpu.async_copy(
          rows_v.at[b], out_hbm.at[pl.ds(base + c * _CHUNK, _CHUNK)],
          ssem[b])

    for c in range(min(_NIBUF, _NCHUNK)):
      start_idx(c)

    for c in range(_NCHUNK):
      b = c % _NBUF
      if c >= _NBUF:
        sd[c - _NBUF].wait()
      idxd[c].wait()
      gd[c] = pltpu.async_copy(
          table_hbm.at[idx_v.at[c % _NIBUF]], rows_v.at[b], gsem[b])
      if c >= _GDEPTH:
        gd[c - _GDEPTH].wait()
        start_store(c - _GDEPTH)
        nxt = c - _GDEPTH + _NIBUF
        if _NIBUF <= nxt < _NCHUNK:
          start_idx(nxt)

    for c in range(_NCHUNK - _GDEPTH, _NCHUNK):
      gd[c].wait()
      start_store(c)
    for c in range(max(0, _NCHUNK - _NBUF), _NCHUNK):
      sd[c].wait()

  return gather_k


_gather = _make_gather()


@jax.jit
def kernel(x, embedding_table):
  idx = x.reshape(-1).astype(jnp.int32)
  out = _gather(embedding_table, idx)
  return out.reshape(x.shape[0], x.shape[1], EMBED)
